# Initial kernel scaffold; baseline (speedup 1.0000x reference)
import jax
import jax.numpy as jnp
from jax.experimental import pallas as pl


def kernel(src_features, neighbor_features, agg_weight, self_weight):
    raise NotImplementedError("write your pallas kernel here")


# speedup vs baseline: 1.3900x; 1.3900x over previous
import functools

import jax
import jax.numpy as jnp
from jax.experimental import pallas as pl


def _fused_body(neigh_ref, src_ref, wa_ref, ws_ref, out_ref):
    s = jnp.sum(neigh_ref[...], axis=1) * (1.0 / neigh_ref.shape[1])
    h = jnp.dot(s, wa_ref[...], preferred_element_type=jnp.float32)
    h += jnp.dot(src_ref[...], ws_ref[...], preferred_element_type=jnp.float32)
    out_ref[...] = jnp.maximum(h, 0.0)


@functools.partial(jax.jit, static_argnames=("row_block",))
def _fused(src_features, neighbor_features, agg_weight, self_weight, row_block=1000):
    n, deg, d_in = neighbor_features.shape
    d_hid = agg_weight.shape[1]
    grid = (n // row_block,)
    return pl.pallas_call(
        _fused_body,
        grid=grid,
        in_specs=[
            pl.BlockSpec((row_block, deg, d_in), lambda i: (i, 0, 0)),
            pl.BlockSpec((row_block, d_in), lambda i: (i, 0)),
            pl.BlockSpec((d_in, d_hid), lambda i: (0, 0)),
            pl.BlockSpec((d_in, d_hid), lambda i: (0, 0)),
        ],
        out_specs=pl.BlockSpec((row_block, d_hid), lambda i: (i, 0)),
        out_shape=jax.ShapeDtypeStruct((n, d_hid), jnp.float32),
    )(neighbor_features, src_features, agg_weight, self_weight)


def kernel(src_features, neighbor_features, agg_weight, self_weight):
    return _fused(src_features, neighbor_features, agg_weight, self_weight)
